# Initial kernel scaffold; baseline (speedup 1.0000x reference)
import functools

import jax
import jax.numpy as jnp
from jax import lax
from jax.experimental import pallas as pl
from jax.experimental.pallas import tpu as pltpu
from jax.experimental.pallas import tpu_sc as plsc

N = 10000
D = 128
G = 64
LANES = 16

NC = 2
NS = 16
NW = NC * NS
CHUNK = 128


def _sc_edge_kernel(n_chunks, npad,
                    hin_hbm, src_hbm, dst_hbm, attr_hbm, w_hbm, b_hbm,
                    out_hbm,
                    src_v, dst_v, attr_v, rows_v, w_v, b_v, aggr_sh, sem):
    c = lax.axis_index("c")
    s = lax.axis_index("s")
    wid = c * NS + s

    pltpu.sync_copy(w_hbm, w_v)
    pltpu.sync_copy(b_hbm, b_v)
    pltpu.sync_copy(src_hbm.at[wid], src_v)
    pltpu.sync_copy(dst_hbm.at[wid], dst_v)
    pltpu.sync_copy(attr_hbm.at[wid], attr_v)

    zeros16 = jnp.zeros((LANES,), jnp.float32)

    def zero_row(i, carry):
        for j in range(D // LANES):
            rows_v[i, pl.ds(j * LANES, LANES)] = zeros16
        return carry

    lax.fori_loop(0, CHUNK, zero_row, 0)

    stripe = npad // NS
    zbase = s * stripe
    zfull, zrem = divmod(stripe, CHUNK)
    for t in range(zfull):
        pltpu.sync_copy(rows_v, aggr_sh.at[pl.ds(zbase + t * CHUNK, CHUNK)])
    if zrem:
        pltpu.sync_copy(rows_v.at[pl.ds(0, zrem)],
                        aggr_sh.at[pl.ds(zbase + zfull * CHUNK, zrem)])

    plsc.subcore_barrier()

    wvec = [[w_v[k, pl.ds(j * LANES, LANES)] for j in range(D // LANES)]
            for k in range(4)]
    bvec = [b_v[pl.ds(j * LANES, LANES)] for j in range(D // LANES)]

    def chunk_body(j, carry):
        pltpu.async_copy(hin_hbm.at[src_v.at[j]], rows_v, sem).wait()

        def edge_body(e, ecarry):
            a0 = attr_v[j * CHUNK + e, 0]
            a1 = attr_v[j * CHUNK + e, 1]
            a2 = attr_v[j * CHUNK + e, 2]
            a3 = attr_v[j * CHUNK + e, 3]
            for jj in range(D // LANES):
                r = rows_v[e, pl.ds(jj * LANES, LANES)]
                emb = (bvec[jj] + a0 * wvec[0][jj] + a1 * wvec[1][jj]
                       + a2 * wvec[2][jj] + a3 * wvec[3][jj])
                rows_v[e, pl.ds(jj * LANES, LANES)] = jnp.maximum(r + emb, 0.0)
            return ecarry

        lax.fori_loop(0, CHUNK, edge_body, 0)

        pltpu.sync_copy(rows_v, aggr_sh.at[dst_v.at[j]], add=True)
        return carry

    lax.fori_loop(0, n_chunks, chunk_body, 0)

    plsc.subcore_barrier()

    obase = s * (N // NS)
    ofull, orem = divmod(N // NS, CHUNK)
    for t in range(ofull):
        pltpu.sync_copy(aggr_sh.at[pl.ds(obase + t * CHUNK, CHUNK)], rows_v)
        pltpu.sync_copy(rows_v, out_hbm.at[c, pl.ds(obase + t * CHUNK, CHUNK)])
    if orem:
        pltpu.sync_copy(aggr_sh.at[pl.ds(obase + ofull * CHUNK, orem)],
                        rows_v.at[pl.ds(0, orem)])
        pltpu.sync_copy(rows_v.at[pl.ds(0, orem)],
                        out_hbm.at[c, pl.ds(obase + ofull * CHUNK, orem)])


def _sc_edge_aggr(hin, src3, dst3, attr3, n_chunks, per_tile, npad, w, b):
    mesh = plsc.VectorSubcoreMesh(core_axis_name="c", subcore_axis_name="s")
    kern = pl.kernel(
        functools.partial(_sc_edge_kernel, n_chunks, npad),
        mesh=mesh,
        out_type=jax.ShapeDtypeStruct((NC, N, D), jnp.float32),
        scratch_types=[
            pltpu.VMEM((n_chunks, CHUNK), jnp.int32),
            pltpu.VMEM((n_chunks, CHUNK), jnp.int32),
            pltpu.VMEM((per_tile, 4), jnp.float32),
            pltpu.VMEM((CHUNK, D), jnp.float32),
            pltpu.VMEM((4, D), jnp.float32),
            pltpu.VMEM((D,), jnp.float32),
            pltpu.VMEM_SHARED((npad, D), jnp.float32),
            pltpu.SemaphoreType.DMA,
        ],
    )
    return kern(hin, src3, dst3, attr3, w, b)


def _bn(h, g, b):
    m = jnp.mean(h, axis=0)
    v = jnp.mean((h - m) ** 2, axis=0)
    return (h - m) * jax.lax.rsqrt(v + 1e-5) * g + b


def _enc_body(x_ref, w_ref, b_ref, o_ref):
    o_ref[...] = (jnp.dot(x_ref[...], w_ref[...],
                          preferred_element_type=jnp.float32) + b_ref[...])


def _tc_encoder(x, w, b):
    return pl.pallas_call(
        _enc_body,
        out_shape=jax.ShapeDtypeStruct((x.shape[0], w.shape[1]), jnp.float32),
    )(x, w, b.reshape(1, -1))


def _layer_body(hin_ref, aggr_ref, batch_ref, vn_ref, epsv_ref,
                wm1_ref, bm1_ref, g1_ref, bt1_ref, wm2_ref, bm2_ref,
                go_ref, bo_ref,
                wv1_ref, bv1_ref, gv1_ref, btv1_ref, wv2_ref, bv2_ref,
                gv2_ref, btv2_ref,
                hnext_ref, vnew_ref):
    h_in = hin_ref[...]
    z = epsv_ref[0, 0] * h_in + aggr_ref[0] + aggr_ref[1]
    t = jnp.dot(z, wm1_ref[...], preferred_element_type=jnp.float32) + bm1_ref[...]
    t = jnp.maximum(_bn(t, g1_ref[...], bt1_ref[...]), 0.0)
    out = jnp.dot(t, wm2_ref[...], preferred_element_type=jnp.float32) + bm2_ref[...]
    h_new = jnp.maximum(_bn(out, go_ref[...], bo_ref[...]), 0.0)

    onehot = (batch_ref[...] ==
              lax.broadcasted_iota(jnp.int32, (N, G), 1)).astype(jnp.float32)
    seg = lax.dot_general(onehot, h_in, (((0,), (0,)), ((), ())),
                          preferred_element_type=jnp.float32)
    vn_tmp = seg + vn_ref[...]
    tv = jnp.dot(vn_tmp, wv1_ref[...], preferred_element_type=jnp.float32) + bv1_ref[...]
    tv = jnp.maximum(_bn(tv, gv1_ref[...], btv1_ref[...]), 0.0)
    tv = jnp.dot(tv, wv2_ref[...], preferred_element_type=jnp.float32) + bv2_ref[...]
    vn_new = jnp.maximum(_bn(tv, gv2_ref[...], btv2_ref[...]), 0.0)

    hnext_ref[...] = h_new + jnp.dot(onehot, vn_new,
                                     preferred_element_type=jnp.float32)
    vnew_ref[...] = vn_new


def _tc_layer(h_in, aggr2, batch2, vn, epsv, wm1, bm1, g1, bt1, wm2, bm2,
              go, bo, wv1, bv1, gv1, btv1, wv2, bv2, gv2, btv2):
    return pl.pallas_call(
        _layer_body,
        out_shape=(jax.ShapeDtypeStruct((N, D), jnp.float32),
                   jax.ShapeDtypeStruct((G, D), jnp.float32)),
    )(h_in, aggr2, batch2, vn, epsv,
      wm1, bm1.reshape(1, -1), g1.reshape(1, -1), bt1.reshape(1, -1),
      wm2, bm2.reshape(1, -1), go.reshape(1, -1), bo.reshape(1, -1),
      wv1, bv1.reshape(1, -1), gv1.reshape(1, -1), btv1.reshape(1, -1),
      wv2, bv2.reshape(1, -1), gv2.reshape(1, -1), btv2.reshape(1, -1))


def _final_body(hin_ref, aggr_ref, batch_ref, epsv_ref,
                wm1_ref, bm1_ref, g1_ref, bt1_ref, wm2_ref, bm2_ref,
                go_ref, bo_ref, wp_ref, bp_ref, o_ref):
    h_in = hin_ref[...]
    z = epsv_ref[0, 0] * h_in + aggr_ref[0] + aggr_ref[1]
    t = jnp.dot(z, wm1_ref[...], preferred_element_type=jnp.float32) + bm1_ref[...]
    t = jnp.maximum(_bn(t, g1_ref[...], bt1_ref[...]), 0.0)
    out = jnp.dot(t, wm2_ref[...], preferred_element_type=jnp.float32) + bm2_ref[...]
    h_last = _bn(out, go_ref[...], bo_ref[...])

    onehot = (batch_ref[...] ==
              lax.broadcasted_iota(jnp.int32, (N, G), 1)).astype(jnp.float32)
    sums = lax.dot_general(onehot, h_last, (((0,), (0,)), ((), ())),
                           preferred_element_type=jnp.float32)
    counts = jnp.sum(onehot, axis=0)
    h_graph = sums / jnp.maximum(counts, 1.0)[:, None]
    o_ref[...] = (jnp.dot(h_graph, wp_ref[...],
                          preferred_element_type=jnp.float32) + bp_ref[...])


def _tc_final(h_in, aggr2, batch2, epsv, wm1, bm1, g1, bt1, wm2, bm2, go, bo,
              wp, bp):
    return pl.pallas_call(
        _final_body,
        out_shape=jax.ShapeDtypeStruct((G, wp.shape[1]), jnp.float32),
    )(h_in, aggr2, batch2, epsv,
      wm1, bm1.reshape(1, -1), g1.reshape(1, -1), bt1.reshape(1, -1),
      wm2, bm2.reshape(1, -1), go.reshape(1, -1), bo.reshape(1, -1),
      wp, bp.reshape(1, -1))


def kernel(x, edge_index, edge_attr, batch, W_enc, b_enc, We_edge, be_edge,
           eps, Wm1, bm1, g1, bt1, Wm2, bm2, g_out, b_out,
           Wv1, bv1, gv1, btv1, Wv2, bv2, gv2, btv2, W_pred, b_pred):
    L = Wm1.shape[0]
    E = edge_index.shape[1]

    per_tile = -(-E // (NW * CHUNK)) * CHUNK
    n_chunks = per_tile // CHUNK
    epad = NW * per_tile
    npad = N + NS
    src = edge_index[0]
    dst = edge_index[1]
    pad = epad - E
    src3 = jnp.concatenate(
        [src, jnp.zeros((pad,), jnp.int32)]).reshape(NW, n_chunks, CHUNK)
    dst3 = jnp.concatenate(
        [dst, jnp.full((pad,), N, jnp.int32)]).reshape(NW, n_chunks, CHUNK)
    attr3 = jnp.concatenate(
        [edge_attr, jnp.zeros((pad, edge_attr.shape[1]), jnp.float32)]
    ).reshape(NW, per_tile, edge_attr.shape[1])

    batch2 = batch.reshape(N, 1)

    h_in = _tc_encoder(x, W_enc, b_enc)
    vn = jnp.zeros((G, D), jnp.float32)

    out = None
    for l in range(L):
        aggr2 = _sc_edge_aggr(h_in, src3, dst3, attr3, n_chunks, per_tile,
                              npad, We_edge[l], be_edge[l])
        epsv = (1.0 + eps[l]).reshape(1, 1)
        if l < L - 1:
            h_in, vn = _tc_layer(
                h_in, aggr2, batch2, vn, epsv,
                Wm1[l], bm1[l], g1[l], bt1[l], Wm2[l], bm2[l],
                g_out[l], b_out[l],
                Wv1[l], bv1[l], gv1[l], btv1[l], Wv2[l], bv2[l],
                gv2[l], btv2[l])
        else:
            out = _tc_final(h_in, aggr2, batch2, epsv,
                            Wm1[l], bm1[l], g1[l], bt1[l], Wm2[l], bm2[l],
                            g_out[l], b_out[l], W_pred, b_pred)
    return out


# speedup vs baseline: 1.9396x; 1.9396x over previous
import functools

import jax
import jax.numpy as jnp
from jax import lax
from jax.experimental import pallas as pl
from jax.experimental.pallas import tpu as pltpu
from jax.experimental.pallas import tpu_sc as plsc

N = 10000
D = 128
G = 64
LANES = 16

NC = 2
NS = 16
NW = NC * NS
CHUNK = 128


def _sc_edge_kernel(n_chunks, npad,
                    hin_hbm, src_hbm, dst_hbm, attr_hbm, w_hbm, b_hbm,
                    out_hbm,
                    src_v, dst_v, attr_v, rows_v, w_v, b_v, aggr_sh, sem):
    c = lax.axis_index("c")
    s = lax.axis_index("s")
    wid = c * NS + s

    pltpu.sync_copy(w_hbm, w_v)
    pltpu.sync_copy(b_hbm, b_v)
    pltpu.sync_copy(src_hbm.at[wid], src_v)
    pltpu.sync_copy(dst_hbm.at[wid], dst_v)
    pltpu.sync_copy(attr_hbm.at[wid], attr_v)

    zeros16 = jnp.zeros((LANES,), jnp.float32)

    def zero_row(i, carry):
        for j in range(D // LANES):
            rows_v[i, pl.ds(j * LANES, LANES)] = zeros16
        return carry

    lax.fori_loop(0, CHUNK, zero_row, 0)

    stripe = npad // NS
    zbase = s * stripe
    zfull, zrem = divmod(stripe, CHUNK)
    for t in range(zfull):
        pltpu.sync_copy(rows_v, aggr_sh.at[pl.ds(zbase + t * CHUNK, CHUNK), :])
    if zrem:
        pltpu.sync_copy(rows_v.at[pl.ds(0, zrem), :],
                        aggr_sh.at[pl.ds(zbase + zfull * CHUNK, zrem), :])

    plsc.subcore_barrier()

    wvec = [[w_v[k, pl.ds(j * LANES, LANES)] for j in range(D // LANES)]
            for k in range(4)]
    bvec = [b_v[pl.ds(j * LANES, LANES)] for j in range(D // LANES)]

    def outer_body(j2, carry):
        pltpu.sync_copy(attr_hbm.at[wid * (n_chunks // 2) + j2], attr_v)
        for h in range(2):
            j = j2 * 2 + h
            pltpu.async_copy(hin_hbm.at[src_v.at[j]], rows_v, sem).wait()

            def quad_body(q, qcarry):
                row = h * 4 + (q >> 3)
                lane = pl.multiple_of((q & 7) * LANES, LANES)
                avec = attr_v[row, pl.ds(lane, LANES)]
                for i in range(4):
                    e = q * 4 + i
                    a0 = avec[4 * i]
                    a1 = avec[4 * i + 1]
                    a2 = avec[4 * i + 2]
                    a3 = avec[4 * i + 3]
                    for jj in range(D // LANES):
                        r = rows_v[e, pl.ds(jj * LANES, LANES)]
                        emb = (bvec[jj] + a0 * wvec[0][jj] + a1 * wvec[1][jj]
                               + a2 * wvec[2][jj] + a3 * wvec[3][jj])
                        rows_v[e, pl.ds(jj * LANES, LANES)] = jnp.maximum(
                            r + emb, 0.0)
                return qcarry

            lax.fori_loop(0, CHUNK // 4, quad_body, 0)

            pltpu.sync_copy(rows_v, aggr_sh.at[dst_v.at[j]], add=True)
        return carry

    lax.fori_loop(0, n_chunks // 2, outer_body, 0)

    plsc.subcore_barrier()

    orows = -(-N // NS) + 7 & ~7
    obase = pl.multiple_of(jnp.minimum(s * orows, N - orows), 8)
    ofull, orem = divmod(orows, CHUNK)
    for t in range(ofull):
        pltpu.sync_copy(aggr_sh.at[pl.ds(obase + t * CHUNK, CHUNK), :], rows_v)
        pltpu.sync_copy(
            rows_v,
            out_hbm.at[pl.ds(pl.multiple_of(c * N + obase + t * CHUNK, 8),
                             CHUNK), :])
    if orem:
        pltpu.sync_copy(aggr_sh.at[pl.ds(obase + ofull * CHUNK, orem), :],
                        rows_v.at[pl.ds(0, orem), :])
        pltpu.sync_copy(
            rows_v.at[pl.ds(0, orem), :],
            out_hbm.at[pl.ds(pl.multiple_of(c * N + obase + ofull * CHUNK, 8),
                             orem), :])


def _sc_edge_aggr(hin, src3, dst3, attr3, n_chunks, per_tile, npad, w, b):
    mesh = plsc.VectorSubcoreMesh(core_axis_name="c", subcore_axis_name="s")
    kern = pl.kernel(
        functools.partial(_sc_edge_kernel, n_chunks, npad),
        mesh=mesh,
        out_type=jax.ShapeDtypeStruct((NC * N, D), jnp.float32),
        scratch_types=[
            pltpu.VMEM((n_chunks, CHUNK), jnp.int32),
            pltpu.VMEM((n_chunks, CHUNK), jnp.int32),
            pltpu.VMEM((8, 128), jnp.float32),
            pltpu.VMEM((CHUNK, D), jnp.float32),
            pltpu.VMEM((4, D), jnp.float32),
            pltpu.VMEM((D,), jnp.float32),
            pltpu.VMEM_SHARED((npad, D), jnp.float32),
            pltpu.SemaphoreType.DMA,
        ],
    )
    return kern(hin, src3, dst3, attr3, w, b).reshape(NC, N, D)


def _bn(h, g, b):
    m = jnp.mean(h, axis=0)
    v = jnp.mean((h - m) ** 2, axis=0)
    return (h - m) * jax.lax.rsqrt(v + 1e-5) * g + b


def _enc_body(x_ref, w_ref, b_ref, o_ref):
    o_ref[...] = (jnp.dot(x_ref[...], w_ref[...],
                          preferred_element_type=jnp.float32, precision=lax.Precision.HIGHEST) + b_ref[...])


def _tc_encoder(x, w, b):
    return pl.pallas_call(
        _enc_body,
        out_shape=jax.ShapeDtypeStruct((x.shape[0], w.shape[1]), jnp.float32),
    )(x, w, b.reshape(1, -1))


def _layer_body(hin_ref, aggr_ref, batch_ref, vn_ref, epsv_ref,
                wm1_ref, bm1_ref, g1_ref, bt1_ref, wm2_ref, bm2_ref,
                go_ref, bo_ref,
                wv1_ref, bv1_ref, gv1_ref, btv1_ref, wv2_ref, bv2_ref,
                gv2_ref, btv2_ref,
                hnext_ref, vnew_ref):
    h_in = hin_ref[...]
    z = epsv_ref[0, 0] * h_in + aggr_ref[0] + aggr_ref[1]
    t = jnp.dot(z, wm1_ref[...], preferred_element_type=jnp.float32, precision=lax.Precision.HIGHEST) + bm1_ref[...]
    t = jnp.maximum(_bn(t, g1_ref[...], bt1_ref[...]), 0.0)
    out = jnp.dot(t, wm2_ref[...], preferred_element_type=jnp.float32, precision=lax.Precision.HIGHEST) + bm2_ref[...]
    h_new = jnp.maximum(_bn(out, go_ref[...], bo_ref[...]), 0.0)

    onehot = (batch_ref[...] ==
              lax.broadcasted_iota(jnp.int32, (N, G), 1)).astype(jnp.float32)
    seg = lax.dot_general(onehot, h_in, (((0,), (0,)), ((), ())),
                          preferred_element_type=jnp.float32, precision=lax.Precision.HIGHEST)
    vn_tmp = seg + vn_ref[...]
    tv = jnp.dot(vn_tmp, wv1_ref[...], preferred_element_type=jnp.float32, precision=lax.Precision.HIGHEST) + bv1_ref[...]
    tv = jnp.maximum(_bn(tv, gv1_ref[...], btv1_ref[...]), 0.0)
    tv = jnp.dot(tv, wv2_ref[...], preferred_element_type=jnp.float32, precision=lax.Precision.HIGHEST) + bv2_ref[...]
    vn_new = jnp.maximum(_bn(tv, gv2_ref[...], btv2_ref[...]), 0.0)

    hnext_ref[...] = h_new + jnp.dot(onehot, vn_new,
                                     preferred_element_type=jnp.float32, precision=lax.Precision.HIGHEST)
    vnew_ref[...] = vn_new


def _tc_layer(h_in, aggr2, batch2, vn, epsv, wm1, bm1, g1, bt1, wm2, bm2,
              go, bo, wv1, bv1, gv1, btv1, wv2, bv2, gv2, btv2):
    return pl.pallas_call(
        _layer_body,
        out_shape=(jax.ShapeDtypeStruct((N, D), jnp.float32),
                   jax.ShapeDtypeStruct((G, D), jnp.float32)),
    )(h_in, aggr2, batch2, vn, epsv,
      wm1, bm1.reshape(1, -1), g1.reshape(1, -1), bt1.reshape(1, -1),
      wm2, bm2.reshape(1, -1), go.reshape(1, -1), bo.reshape(1, -1),
      wv1, bv1.reshape(1, -1), gv1.reshape(1, -1), btv1.reshape(1, -1),
      wv2, bv2.reshape(1, -1), gv2.reshape(1, -1), btv2.reshape(1, -1))


def _final_body(hin_ref, aggr_ref, batch_ref, epsv_ref,
                wm1_ref, bm1_ref, g1_ref, bt1_ref, wm2_ref, bm2_ref,
                go_ref, bo_ref, wp_ref, bp_ref, o_ref):
    h_in = hin_ref[...]
    z = epsv_ref[0, 0] * h_in + aggr_ref[0] + aggr_ref[1]
    t = jnp.dot(z, wm1_ref[...], preferred_element_type=jnp.float32, precision=lax.Precision.HIGHEST) + bm1_ref[...]
    t = jnp.maximum(_bn(t, g1_ref[...], bt1_ref[...]), 0.0)
    out = jnp.dot(t, wm2_ref[...], preferred_element_type=jnp.float32, precision=lax.Precision.HIGHEST) + bm2_ref[...]
    h_last = _bn(out, go_ref[...], bo_ref[...])

    onehot = (batch_ref[...] ==
              lax.broadcasted_iota(jnp.int32, (N, G), 1)).astype(jnp.float32)
    sums = lax.dot_general(onehot, h_last, (((0,), (0,)), ((), ())),
                           preferred_element_type=jnp.float32, precision=lax.Precision.HIGHEST)
    counts = jnp.sum(onehot, axis=0)
    h_graph = sums / jnp.maximum(counts, 1.0)[:, None]
    o_ref[...] = (jnp.dot(h_graph, wp_ref[...],
                          preferred_element_type=jnp.float32, precision=lax.Precision.HIGHEST) + bp_ref[...])


def _tc_final(h_in, aggr2, batch2, epsv, wm1, bm1, g1, bt1, wm2, bm2, go, bo,
              wp, bp):
    return pl.pallas_call(
        _final_body,
        out_shape=jax.ShapeDtypeStruct((G, wp.shape[1]), jnp.float32),
    )(h_in, aggr2, batch2, epsv,
      wm1, bm1.reshape(1, -1), g1.reshape(1, -1), bt1.reshape(1, -1),
      wm2, bm2.reshape(1, -1), go.reshape(1, -1), bo.reshape(1, -1),
      wp, bp.reshape(1, -1))


def kernel(x, edge_index, edge_attr, batch, W_enc, b_enc, We_edge, be_edge,
           eps, Wm1, bm1, g1, bt1, Wm2, bm2, g_out, b_out,
           Wv1, bv1, gv1, btv1, Wv2, bv2, gv2, btv2, W_pred, b_pred):
    L = Wm1.shape[0]
    E = edge_index.shape[1]

    per_tile = -(-E // (NW * 2 * CHUNK)) * 2 * CHUNK
    n_chunks = per_tile // CHUNK
    epad = NW * per_tile
    npad = 10112
    src = edge_index[0]
    dst = edge_index[1]
    pad = epad - E
    src3 = jnp.concatenate(
        [src, jnp.zeros((pad,), jnp.int32)]).reshape(NW, n_chunks, CHUNK)
    dst3 = jnp.concatenate(
        [dst, jnp.full((pad,), N, jnp.int32)]).reshape(NW, n_chunks, CHUNK)
    attr3 = jnp.concatenate(
        [edge_attr, jnp.zeros((pad, edge_attr.shape[1]), jnp.float32)]
    ).reshape(NW * (per_tile // 256), 8, 32 * edge_attr.shape[1])

    batch2 = batch.reshape(N, 1)

    h_in = _tc_encoder(x, W_enc, b_enc)
    vn = jnp.zeros((G, D), jnp.float32)

    out = None
    for l in range(L):
        aggr2 = _sc_edge_aggr(h_in, src3, dst3, attr3, n_chunks, per_tile,
                              npad, We_edge[l], be_edge[l])
        epsv = (1.0 + eps[l]).reshape(1, 1)
        if l < L - 1:
            h_in, vn = _tc_layer(
                h_in, aggr2, batch2, vn, epsv,
                Wm1[l], bm1[l], g1[l], bt1[l], Wm2[l], bm2[l],
                g_out[l], b_out[l],
                Wv1[l], bv1[l], gv1[l], btv1[l], Wv2[l], bv2[l],
                gv2[l], btv2[l])
        else:
            out = _tc_final(h_in, aggr2, batch2, epsv,
                            Wm1[l], bm1[l], g1[l], bt1[l], Wm2[l], bm2[l],
                            g_out[l], b_out[l], W_pred, b_pred)
    return out
